# baseline (speedup 1.0000x reference)
import jax
import jax.numpy as jnp
from jax.experimental import pallas as pl

N = 16384
K = 1000
F = 16
TN = 512
G = N // TN


def _body(x_ref, c_ref, dist_ref, asg_ref, ine_ref):
    x = x_ref[...]
    c = c_ref[...]
    x2 = jnp.sum(x * x, axis=1, keepdims=True)
    c2 = jnp.sum(c * c, axis=1)[None, :]
    xc = jax.lax.dot_general(
        x, c, (((1,), (1,)), ((), ())), preferred_element_type=jnp.float32
    )
    d2 = jnp.maximum(x2 + c2 - 2.0 * xc, 0.0)
    dist = jnp.sqrt(d2)
    dist_ref[...] = dist
    m = jnp.min(dist, axis=1)
    iota = jax.lax.broadcasted_iota(jnp.int32, dist.shape, 1)
    idx = jnp.min(jnp.where(dist == m[:, None], iota, K), axis=1)
    asg_ref[0, 0, :] = idx
    ine_ref[0, 0, :] = m * m


def kernel(data, centroids):
    distances, asg3, ine3 = pl.pallas_call(
        _body,
        grid=(G,),
        in_specs=[
            pl.BlockSpec((TN, F), lambda i: (i, 0)),
            pl.BlockSpec((K, F), lambda i: (0, 0)),
        ],
        out_specs=[
            pl.BlockSpec((TN, K), lambda i: (i, 0)),
            pl.BlockSpec((1, 1, TN), lambda i: (i, 0, 0)),
            pl.BlockSpec((1, 1, TN), lambda i: (i, 0, 0)),
        ],
        out_shape=[
            jax.ShapeDtypeStruct((N, K), jnp.float32),
            jax.ShapeDtypeStruct((G, 1, TN), jnp.int32),
            jax.ShapeDtypeStruct((G, 1, TN), jnp.float32),
        ],
    )(data, centroids)
    return distances, asg3.reshape(N), ine3.reshape(N)
